# Initial kernel scaffold; baseline (speedup 1.0000x reference)
import jax
import jax.numpy as jnp
from jax.experimental import pallas as pl


def kernel(edge_index, N, y, emb, W1, b1, W2, b2, Wout, bout):
    raise NotImplementedError("write your pallas kernel here")


# speedup vs baseline: 5.6838x; 5.6838x over previous
import functools

import jax
import jax.numpy as jnp
from jax import lax
from jax.experimental import pallas as pl
from jax.experimental.pallas import tpu as pltpu
from jax.experimental.pallas import tpu_sc as plsc

NN = 10000
NNP = 10240
E = 320000
D = 128
V = 100000
T = 128
NC, NS = 2, 16
NW = NC * NS
FPW = D // NW
BPW = NNP // NW
GCH = 64
NGC = BPW // GCH
EPW = E // NW
ECH = 16000
NEC = E // ECH

_mesh = plsc.VectorSubcoreMesh(
    core_axis_name="c", subcore_axis_name="s", num_cores=NC, num_subcores=NS
)
_sc_params = pltpu.CompilerParams(needs_layout_passes=False)


def _wid():
    return lax.axis_index("s") * NC + lax.axis_index("c")


def _prep_body(emb, y_r, dst_e, zeros, rows_out, degp_out,
               idx_v, rows_v, dst_v, deg_v, sem):
    w = _wid()
    base = w * BPW
    pltpu.sync_copy(y_r.at[w], idx_v)
    for q in range(NGC):
        pltpu.async_copy(emb.at[idx_v.at[q]],
                         rows_v.at[pl.ds(q * GCH, GCH)], sem).wait()
    pltpu.sync_copy(rows_v, rows_out.at[pl.ds(base, BPW)])
    pltpu.sync_copy(zeros.at[pl.ds(0, NNP)], deg_v)
    pltpu.sync_copy(dst_e.at[pl.ds(w * EPW, EPW)], dst_v)
    ones = jnp.ones((16,), jnp.float32)

    def body(g, carry):
        d16 = dst_v[pl.ds(g * 16, 16)]
        plsc.addupdate_scatter(deg_v, [d16], ones)
        return carry

    lax.fori_loop(0, EPW // 16, body, 0)
    pltpu.sync_copy(deg_v, degp_out.at[pl.ds(w * NNP, NNP)])


_prep = functools.partial(
    pl.kernel,
    out_type=[
        jax.ShapeDtypeStruct((NNP, D), jnp.float32),
        jax.ShapeDtypeStruct((NW * NNP,), jnp.float32),
    ],
    mesh=_mesh,
    scratch_types=[
        pltpu.VMEM((NGC, GCH), jnp.int32),
        pltpu.VMEM((BPW, D), jnp.float32),
        pltpu.VMEM((EPW,), jnp.int32),
        pltpu.VMEM((NNP,), jnp.float32),
        pltpu.SemaphoreType.DMA,
    ],
    compiler_params=_sc_params,
)(_prep_body)


def _spmm_body(sel_only, xw_t, src_e, dst_e, dinv, zeros, n_idx, out,
               xs, acc, dinv_v, src_v, dst_v, n_v, sel_v):
    w = _wid()
    pltpu.sync_copy(xw_t.at[pl.ds(w * FPW * NNP, FPW * NNP)], xs)
    pltpu.sync_copy(dinv, dinv_v)
    pltpu.sync_copy(zeros, acc)

    def chunk(k, carry):
        pltpu.sync_copy(src_e.at[pl.ds(k * ECH, ECH)], src_v)
        pltpu.sync_copy(dst_e.at[pl.ds(k * ECH, ECH)], dst_v)

        def group(g, c2):
            off = g * 16
            s16 = src_v[pl.ds(off, 16)]
            d16 = dst_v[pl.ds(off, 16)]
            nv = (plsc.load_gather(dinv_v, [s16])
                  * plsc.load_gather(dinv_v, [d16]))
            for j in range(FPW):
                xv = plsc.load_gather(xs, [s16 + (j * NNP)])
                plsc.addupdate_scatter(acc, [d16 + (j * NNP)], xv * nv)
            return c2

        lax.fori_loop(0, ECH // 16, group, 0)
        return carry

    lax.fori_loop(0, NEC, chunk, 0)

    if sel_only:
        pltpu.sync_copy(n_idx, n_v)
        for g in range(T // 16):
            t16 = n_v[pl.ds(g * 16, 16)]
            dv = plsc.load_gather(dinv_v, [t16])
            d2 = dv * dv
            for j in range(FPW):
                av = plsc.load_gather(acc, [t16 + (j * NNP)])
                xv = plsc.load_gather(xs, [t16 + (j * NNP)])
                sel_v[pl.ds(j * T + g * 16, 16)] = av + d2 * xv
        pltpu.sync_copy(sel_v, out.at[pl.ds(w * FPW * T, FPW * T)])
    else:
        def ep(g, carry):
            off = g * 16
            dv = dinv_v[pl.ds(off, 16)]
            d2 = dv * dv
            for j in range(FPW):
                o = j * NNP + off
                acc[pl.ds(o, 16)] = acc[pl.ds(o, 16)] + d2 * xs[pl.ds(o, 16)]
            return carry

        lax.fori_loop(0, NNP // 16, ep, 0)
        pltpu.sync_copy(acc, out.at[pl.ds(w * FPW * NNP, FPW * NNP)])


def _make_spmm(sel_only):
    out_shape = (NW * FPW * T,) if sel_only else (D * NNP,)
    return functools.partial(
        pl.kernel,
        out_type=jax.ShapeDtypeStruct(out_shape, jnp.float32),
        mesh=_mesh,
        scratch_types=[
            pltpu.VMEM((FPW * NNP,), jnp.float32),
            pltpu.VMEM((FPW * NNP,), jnp.float32),
            pltpu.VMEM((NNP,), jnp.float32),
            pltpu.VMEM((ECH,), jnp.int32),
            pltpu.VMEM((ECH,), jnp.int32),
            pltpu.VMEM((T,), jnp.int32),
            pltpu.VMEM((FPW * T,), jnp.float32),
        ],
        compiler_params=_sc_params,
    )(functools.partial(_spmm_body, sel_only))


_spmm_full = _make_spmm(False)
_spmm_sel = _make_spmm(True)


def _ln_w1_body(rows_ref, w1_ref, out_ref):
    r = rows_ref[:]
    mu = jnp.mean(r, axis=-1, keepdims=True)
    var = jnp.mean((r - mu) ** 2, axis=-1, keepdims=True)
    x = (r - mu) * lax.rsqrt(var + 1e-5)
    out_ref[:] = lax.dot_general(
        w1_ref[:], x, (((1,), (1,)), ((), ())),
        preferred_element_type=jnp.float32)


def _dinv_body(degp_ref, out_ref):
    deg = jnp.sum(degp_ref[:], axis=0) + 1.0
    out_ref[:] = lax.rsqrt(deg)


def _act_w2_body(a_ref, b1_ref, w2_ref, out_ref):
    hb = a_ref[:] + b1_ref[:]
    hb = jnp.where(hb > 0, hb, 0.15 * hb)
    out_ref[:] = lax.dot_general(
        w2_ref[:], hb, (((1,), (0,)), ((), ())),
        preferred_element_type=jnp.float32)


def _head_body(sel_ref, b2_ref, wout_ref, bout_ref, out_ref):
    a = sel_ref[:] + b2_ref[:]
    out_ref[:] = lax.dot_general(
        a, wout_ref[:], (((0,), (1,)), ((), ())),
        preferred_element_type=jnp.float32) + bout_ref[:]


_NB = 1024
_VB = 2048


def kernel(edge_index, N, y, emb, W1, b1, W2, b2, Wout, bout):
    src = edge_index[0].astype(jnp.int32)
    dst = edge_index[1].astype(jnp.int32)
    y_pad = jnp.concatenate(
        [y.astype(jnp.int32), jnp.zeros((NNP - NN,), jnp.int32)]
    ).reshape(NW, NGC, GCH)
    n_idx = N.astype(jnp.int32)
    zeros = jnp.zeros((FPW * NNP,), jnp.float32)

    rows, deg_p = _prep(emb, y_pad, dst, zeros)

    xw1_t = pl.pallas_call(
        _ln_w1_body,
        grid=(NNP // _NB,),
        in_specs=[
            pl.BlockSpec((_NB, D), lambda i: (i, 0)),
            pl.BlockSpec((D, D), lambda i: (0, 0)),
        ],
        out_specs=pl.BlockSpec((D, _NB), lambda i: (0, i)),
        out_shape=jax.ShapeDtypeStruct((D, NNP), jnp.float32),
    )(rows, W1)

    dinv = pl.pallas_call(
        _dinv_body,
        in_specs=[pl.BlockSpec((NW, NNP // D, D), lambda: (0, 0, 0))],
        out_specs=pl.BlockSpec((NNP // D, D), lambda: (0, 0)),
        out_shape=jax.ShapeDtypeStruct((NNP // D, D), jnp.float32),
    )(deg_p.reshape(NW, NNP // D, D)).reshape(NNP)

    a1_t = _spmm_full(xw1_t.reshape(D * NNP), src, dst, dinv, zeros, n_idx)

    hw2_t = pl.pallas_call(
        _act_w2_body,
        grid=(NNP // _NB,),
        in_specs=[
            pl.BlockSpec((D, _NB), lambda i: (0, i)),
            pl.BlockSpec((D, 1), lambda i: (0, 0)),
            pl.BlockSpec((D, D), lambda i: (0, 0)),
        ],
        out_specs=pl.BlockSpec((D, _NB), lambda i: (0, i)),
        out_shape=jax.ShapeDtypeStruct((D, NNP), jnp.float32),
    )(a1_t.reshape(D, NNP), b1.reshape(D, 1), W2)

    sel = _spmm_sel(hw2_t.reshape(D * NNP), src, dst, dinv, zeros, n_idx)

    out = pl.pallas_call(
        _head_body,
        grid=(pl.cdiv(V, _VB),),
        in_specs=[
            pl.BlockSpec((D, T), lambda i: (0, 0)),
            pl.BlockSpec((D, 1), lambda i: (0, 0)),
            pl.BlockSpec((_VB, D), lambda i: (i, 0)),
            pl.BlockSpec((1, _VB), lambda i: (0, i)),
        ],
        out_specs=pl.BlockSpec((T, _VB), lambda i: (0, i)),
        out_shape=jax.ShapeDtypeStruct((T, V), jnp.float32),
    )(sel.reshape(NW * FPW, T).reshape(D, T), b2.reshape(D, 1),
      Wout, bout.reshape(1, V))

    return out


# speedup vs baseline: 12.9239x; 2.2738x over previous
import functools

import jax
import jax.numpy as jnp
from jax import lax
from jax.experimental import pallas as pl
from jax.experimental.pallas import tpu as pltpu
from jax.experimental.pallas import tpu_sc as plsc

NN = 10000
NNP = 10240
E = 320000
D = 128
V = 100000
T = 128
NC, NS = 2, 16
NW = NC * NS
FPW = D // NW
BPW = NNP // NW
GCH = 64
NGC = BPW // GCH
EPW = E // NW
ECH = 8000
NEC = E // ECH

_mesh = plsc.VectorSubcoreMesh(
    core_axis_name="c", subcore_axis_name="s", num_cores=NC, num_subcores=NS
)
_sc_params = pltpu.CompilerParams(needs_layout_passes=False)


def _wid():
    return lax.axis_index("s") * NC + lax.axis_index("c")


def _prep_body(emb, y_r, dst_e, zeros, rows_out, degp_out,
               idx_v, rows_v, dst_v, deg_v, sem):
    w = _wid()
    base = w * BPW
    pltpu.sync_copy(y_r.at[w], idx_v)
    for q in range(NGC):
        pltpu.async_copy(emb.at[idx_v.at[q]],
                         rows_v.at[pl.ds(q * GCH, GCH)], sem).wait()
    pltpu.sync_copy(rows_v, rows_out.at[pl.ds(base, BPW)])
    pltpu.sync_copy(zeros.at[pl.ds(0, NNP)], deg_v)
    pltpu.sync_copy(dst_e.at[pl.ds(w * EPW, EPW)], dst_v)
    ones = jnp.ones((16,), jnp.float32)

    @plsc.parallel_loop(0, EPW // 16, unroll=8)
    def _deg(g):
        d16 = dst_v[pl.ds(g * 16, 16)]
        plsc.addupdate_scatter(deg_v, [d16], ones)
    pltpu.sync_copy(deg_v, degp_out.at[pl.ds(w * NNP, NNP)])


_prep = functools.partial(
    pl.kernel,
    out_type=[
        jax.ShapeDtypeStruct((NNP, D), jnp.float32),
        jax.ShapeDtypeStruct((NW * NNP,), jnp.float32),
    ],
    mesh=_mesh,
    scratch_types=[
        pltpu.VMEM((NGC, GCH), jnp.int32),
        pltpu.VMEM((BPW, D), jnp.float32),
        pltpu.VMEM((EPW,), jnp.int32),
        pltpu.VMEM((NNP,), jnp.float32),
        pltpu.SemaphoreType.DMA,
    ],
    compiler_params=_sc_params,
)(_prep_body)


def _spmm_body(sel_only, xw_t, src_e, dst_e, dinv, zeros, n_idx, out,
               xs, acc, dinv_v, src_v0, src_v1, dst_v0, dst_v1,
               n_v, sel_v, sems, semd):
    w = _wid()
    pltpu.sync_copy(xw_t.at[pl.ds(w * FPW * NNP, FPW * NNP)], xs)
    pltpu.sync_copy(dinv, dinv_v)
    pltpu.sync_copy(zeros, acc)

    bufs = ((src_v0, dst_v0), (src_v1, dst_v1))
    for b in range(2):
        pltpu.async_copy(src_e.at[pl.ds(b * ECH, ECH)], bufs[b][0],
                         sems.at[b])
        pltpu.async_copy(dst_e.at[pl.ds(b * ECH, ECH)], bufs[b][1],
                         semd.at[b])

    @pl.loop(0, NEC, step=2)
    def _chunks(k):
        for b in range(2):
            kk = k + b
            sv, dv_ = bufs[b]
            pltpu.make_async_copy(src_e.at[pl.ds(kk * ECH, ECH)],
                                  sv, sems.at[b]).wait()
            pltpu.make_async_copy(dst_e.at[pl.ds(kk * ECH, ECH)],
                                  dv_, semd.at[b]).wait()

            @plsc.parallel_loop(0, ECH // 16, unroll=8)
            def _group(g):
                off = g * 16
                s16 = sv[pl.ds(off, 16)]
                d16 = dv_[pl.ds(off, 16)]
                nv = (plsc.load_gather(dinv_v, [s16])
                      * plsc.load_gather(dinv_v, [d16]))
                for j in range(FPW):
                    xv = plsc.load_gather(xs, [s16 + (j * NNP)])
                    plsc.addupdate_scatter(acc, [d16 + (j * NNP)], xv * nv)

            @pl.when(kk + 2 < NEC)
            def _prefetch():
                pltpu.async_copy(src_e.at[pl.ds((kk + 2) * ECH, ECH)],
                                 sv, sems.at[b])
                pltpu.async_copy(dst_e.at[pl.ds((kk + 2) * ECH, ECH)],
                                 dv_, semd.at[b])

    if sel_only:
        pltpu.sync_copy(n_idx, n_v)
        for g in range(T // 16):
            t16 = n_v[pl.ds(g * 16, 16)]
            dv = plsc.load_gather(dinv_v, [t16])
            d2 = dv * dv
            for j in range(FPW):
                av = plsc.load_gather(acc, [t16 + (j * NNP)])
                xv = plsc.load_gather(xs, [t16 + (j * NNP)])
                sel_v[pl.ds(j * T + g * 16, 16)] = av + d2 * xv
        pltpu.sync_copy(sel_v, out.at[pl.ds(w * FPW * T, FPW * T)])
    else:
        @plsc.parallel_loop(0, NNP // 16, unroll=4)
        def _ep(g):
            off = g * 16
            dv = dinv_v[pl.ds(off, 16)]
            d2 = dv * dv
            for j in range(FPW):
                o = j * NNP + off
                acc[pl.ds(o, 16)] = acc[pl.ds(o, 16)] + d2 * xs[pl.ds(o, 16)]

        pltpu.sync_copy(acc, out.at[pl.ds(w * FPW * NNP, FPW * NNP)])


def _make_spmm(sel_only):
    out_shape = (NW * FPW * T,) if sel_only else (D * NNP,)
    return functools.partial(
        pl.kernel,
        out_type=jax.ShapeDtypeStruct(out_shape, jnp.float32),
        mesh=_mesh,
        scratch_types=[
            pltpu.VMEM((FPW * NNP,), jnp.float32),
            pltpu.VMEM((FPW * NNP,), jnp.float32),
            pltpu.VMEM((NNP,), jnp.float32),
            pltpu.VMEM((ECH,), jnp.int32),
            pltpu.VMEM((ECH,), jnp.int32),
            pltpu.VMEM((ECH,), jnp.int32),
            pltpu.VMEM((ECH,), jnp.int32),
            pltpu.VMEM((T,), jnp.int32),
            pltpu.VMEM((FPW * T,), jnp.float32),
            pltpu.SemaphoreType.DMA((2,)),
            pltpu.SemaphoreType.DMA((2,)),
        ],
        compiler_params=_sc_params,
    )(functools.partial(_spmm_body, sel_only))


_spmm_full = _make_spmm(False)
_spmm_sel = _make_spmm(True)


def _ln_w1_body(rows_ref, w1_ref, out_ref):
    r = rows_ref[:]
    mu = jnp.mean(r, axis=-1, keepdims=True)
    var = jnp.mean((r - mu) ** 2, axis=-1, keepdims=True)
    x = (r - mu) * lax.rsqrt(var + 1e-5)
    out_ref[:] = lax.dot_general(
        w1_ref[:], x, (((1,), (1,)), ((), ())),
        preferred_element_type=jnp.float32)


def _dinv_body(degp_ref, out_ref):
    deg = jnp.sum(degp_ref[:], axis=0) + 1.0
    out_ref[:] = lax.rsqrt(deg)


def _act_w2_body(a_ref, b1_ref, w2_ref, out_ref):
    hb = a_ref[:] + b1_ref[:]
    hb = jnp.where(hb > 0, hb, 0.15 * hb)
    out_ref[:] = lax.dot_general(
        w2_ref[:], hb, (((1,), (0,)), ((), ())),
        preferred_element_type=jnp.float32)


def _head_body(sel_ref, b2_ref, wout_ref, bout_ref, out_ref):
    a = sel_ref[:] + b2_ref[:]
    out_ref[:] = lax.dot_general(
        a, wout_ref[:], (((0,), (1,)), ((), ())),
        preferred_element_type=jnp.float32) + bout_ref[:]


_NB = 1024
_VB = 2048


def kernel(edge_index, N, y, emb, W1, b1, W2, b2, Wout, bout):
    src = edge_index[0].astype(jnp.int32)
    dst = edge_index[1].astype(jnp.int32)
    y_pad = jnp.concatenate(
        [y.astype(jnp.int32), jnp.zeros((NNP - NN,), jnp.int32)]
    ).reshape(NW, NGC, GCH)
    n_idx = N.astype(jnp.int32)
    zeros = jnp.zeros((FPW * NNP,), jnp.float32)

    rows, deg_p = _prep(emb, y_pad, dst, zeros)

    xw1_t = pl.pallas_call(
        _ln_w1_body,
        grid=(NNP // _NB,),
        in_specs=[
            pl.BlockSpec((_NB, D), lambda i: (i, 0)),
            pl.BlockSpec((D, D), lambda i: (0, 0)),
        ],
        out_specs=pl.BlockSpec((D, _NB), lambda i: (0, i)),
        out_shape=jax.ShapeDtypeStruct((D, NNP), jnp.float32),
    )(rows, W1)

    dinv = pl.pallas_call(
        _dinv_body,
        in_specs=[pl.BlockSpec((NW, NNP // D, D), lambda: (0, 0, 0))],
        out_specs=pl.BlockSpec((NNP // D, D), lambda: (0, 0)),
        out_shape=jax.ShapeDtypeStruct((NNP // D, D), jnp.float32),
    )(deg_p.reshape(NW, NNP // D, D)).reshape(NNP)

    a1_t = _spmm_full(xw1_t.reshape(D * NNP), src, dst, dinv, zeros, n_idx)

    hw2_t = pl.pallas_call(
        _act_w2_body,
        grid=(NNP // _NB,),
        in_specs=[
            pl.BlockSpec((D, _NB), lambda i: (0, i)),
            pl.BlockSpec((D, 1), lambda i: (0, 0)),
            pl.BlockSpec((D, D), lambda i: (0, 0)),
        ],
        out_specs=pl.BlockSpec((D, _NB), lambda i: (0, i)),
        out_shape=jax.ShapeDtypeStruct((D, NNP), jnp.float32),
    )(a1_t.reshape(D, NNP), b1.reshape(D, 1), W2)

    sel = _spmm_sel(hw2_t.reshape(D * NNP), src, dst, dinv, zeros, n_idx)

    out = pl.pallas_call(
        _head_body,
        grid=(pl.cdiv(V, _VB),),
        in_specs=[
            pl.BlockSpec((D, T), lambda i: (0, 0)),
            pl.BlockSpec((D, 1), lambda i: (0, 0)),
            pl.BlockSpec((_VB, D), lambda i: (i, 0)),
            pl.BlockSpec((1, _VB), lambda i: (0, i)),
        ],
        out_specs=pl.BlockSpec((T, _VB), lambda i: (0, i)),
        out_shape=jax.ShapeDtypeStruct((T, V), jnp.float32),
    )(sel.reshape(NW * FPW, T).reshape(D, T), b2.reshape(D, 1),
      Wout, bout.reshape(1, V))

    return out


# speedup vs baseline: 18.6828x; 1.4456x over previous
import functools

import jax
import jax.numpy as jnp
from jax import lax
from jax.experimental import pallas as pl
from jax.experimental.pallas import tpu as pltpu
from jax.experimental.pallas import tpu_sc as plsc

NN = 10000
NNP = 10240
E = 320000
D = 128
V = 100000
T = 128
NC, NS = 2, 16
NW = NC * NS
FPW = D // NW
BPW = NNP // NW
GCH = 64
NGC = BPW // GCH
EPW = E // NW
ECH = 8000
NEC = E // ECH
HCAP = 512
PCH = 512
PAD = NNP - 1

_mesh = plsc.VectorSubcoreMesh(
    core_axis_name="c", subcore_axis_name="s", num_cores=NC, num_subcores=NS
)
_sc_params = pltpu.CompilerParams(needs_layout_passes=False)


def _wid():
    return lax.axis_index("s") * NC + lax.axis_index("c")


def _prep_body(emb, y_r, src_e, dst_e, n_idx, zeros,
               rows_out, degp_out, l2s_out, l2d_out, head_out, cnt_out,
               idx_v, rows_v, src_v, dst_v, deg_v, tmask, cs, cd,
               cnt_v, n_v, sem):
    w = _wid()
    base = w * BPW
    pltpu.sync_copy(y_r.at[w], idx_v)
    for q in range(NGC):
        pltpu.async_copy(emb.at[idx_v.at[q]],
                         rows_v.at[pl.ds(q * GCH, GCH)], sem).wait()
    pltpu.sync_copy(rows_v, rows_out.at[pl.ds(base, BPW)])

    pltpu.sync_copy(zeros.at[pl.ds(0, NNP)], tmask)
    pltpu.sync_copy(n_idx, n_v)
    ones = jnp.ones((16,), jnp.float32)
    for g in range(T // 16):
        t16 = n_v[pl.ds(g * 16, 16)]
        plsc.store_scatter(tmask, [t16], ones)

    padv = jnp.full((16,), PAD, jnp.int32)

    @plsc.parallel_loop(0, (EPW + 16) // 16, unroll=8)
    def _fill(g):
        cs[pl.ds(g * 16, 16)] = padv
        cd[pl.ds(g * 16, 16)] = padv

    pltpu.sync_copy(zeros.at[pl.ds(0, NNP)], deg_v)
    pltpu.sync_copy(src_e.at[pl.ds(w * EPW, EPW)], src_v)
    pltpu.sync_copy(dst_e.at[pl.ds(w * EPW, EPW)], dst_v)

    @plsc.parallel_loop(0, EPW // 16, unroll=4, carry=jnp.int32(0))
    def _deg(g, m):
        off = g * 16
        s16 = src_v[pl.ds(off, 16)]
        d16 = dst_v[pl.ds(off, 16)]
        plsc.addupdate_scatter(deg_v, [d16], ones)
        tv = plsc.load_gather(tmask, [d16])
        msk = tv > 0.0
        plsc.store_compressed(cs.at[pl.ds(m, 16)], s16, mask=msk)
        plsc.store_compressed(cd.at[pl.ds(m, 16)], d16, mask=msk)
        return m + jnp.sum(msk.astype(jnp.int32))

    m = _deg
    pltpu.sync_copy(deg_v, degp_out.at[pl.ds(w * NNP, NNP)])
    pltpu.sync_copy(cs.at[pl.ds(0, EPW)], l2s_out.at[pl.ds(w * EPW, EPW)])
    pltpu.sync_copy(cd.at[pl.ds(0, EPW)], l2d_out.at[pl.ds(w * EPW, EPW)])
    hb = w * 2 * HCAP
    pltpu.sync_copy(cs.at[pl.ds(0, HCAP)], head_out.at[pl.ds(hb, HCAP)])
    pltpu.sync_copy(cd.at[pl.ds(0, HCAP)], head_out.at[pl.ds(hb + HCAP, HCAP)])
    iota = lax.iota(jnp.int32, 16)
    cnt_v[...] = jnp.where(iota == 0, m, 0)
    pltpu.sync_copy(cnt_v, cnt_out.at[pl.ds(w * 16, 16)])


_prep = functools.partial(
    pl.kernel,
    out_type=[
        jax.ShapeDtypeStruct((NNP, D), jnp.float32),
        jax.ShapeDtypeStruct((NW * NNP,), jnp.float32),
        jax.ShapeDtypeStruct((NW * EPW,), jnp.int32),
        jax.ShapeDtypeStruct((NW * EPW,), jnp.int32),
        jax.ShapeDtypeStruct((NW * 2 * HCAP,), jnp.int32),
        jax.ShapeDtypeStruct((NW * 16,), jnp.int32),
    ],
    mesh=_mesh,
    scratch_types=[
        pltpu.VMEM((NGC, GCH), jnp.int32),
        pltpu.VMEM((BPW, D), jnp.float32),
        pltpu.VMEM((EPW,), jnp.int32),
        pltpu.VMEM((EPW,), jnp.int32),
        pltpu.VMEM((NNP,), jnp.float32),
        pltpu.VMEM((NNP,), jnp.float32),
        pltpu.VMEM((EPW + 16,), jnp.int32),
        pltpu.VMEM((EPW + 16,), jnp.int32),
        pltpu.VMEM((16,), jnp.int32),
        pltpu.VMEM((T,), jnp.int32),
        pltpu.SemaphoreType.DMA,
    ],
    compiler_params=_sc_params,
)(_prep_body)


def _spmm_body(xw_t, src_e, dst_e, dinv, zeros, out,
               xs, acc, dinv_v, src_v0, src_v1, dst_v0, dst_v1,
               sems, semd):
    w = _wid()
    pltpu.sync_copy(xw_t.at[pl.ds(w * FPW * NNP, FPW * NNP)], xs)
    pltpu.sync_copy(dinv, dinv_v)
    pltpu.sync_copy(zeros, acc)

    bufs = ((src_v0, dst_v0), (src_v1, dst_v1))
    for b in range(2):
        pltpu.async_copy(src_e.at[pl.ds(b * ECH, ECH)], bufs[b][0],
                         sems.at[b])
        pltpu.async_copy(dst_e.at[pl.ds(b * ECH, ECH)], bufs[b][1],
                         semd.at[b])

    @pl.loop(0, NEC, step=2)
    def _chunks(k):
        for b in range(2):
            kk = k + b
            sv, dv_ = bufs[b]
            pltpu.make_async_copy(src_e.at[pl.ds(kk * ECH, ECH)],
                                  sv, sems.at[b]).wait()
            pltpu.make_async_copy(dst_e.at[pl.ds(kk * ECH, ECH)],
                                  dv_, semd.at[b]).wait()

            @plsc.parallel_loop(0, ECH // 16, unroll=8)
            def _group(g):
                off = g * 16
                s16 = sv[pl.ds(off, 16)]
                d16 = dv_[pl.ds(off, 16)]
                nv = (plsc.load_gather(dinv_v, [s16])
                      * plsc.load_gather(dinv_v, [d16]))
                for j in range(FPW):
                    xv = plsc.load_gather(xs, [s16 + (j * NNP)])
                    plsc.addupdate_scatter(acc, [d16 + (j * NNP)], xv * nv)

            @pl.when(kk + 2 < NEC)
            def _prefetch():
                pltpu.async_copy(src_e.at[pl.ds((kk + 2) * ECH, ECH)],
                                 sv, sems.at[b])
                pltpu.async_copy(dst_e.at[pl.ds((kk + 2) * ECH, ECH)],
                                 dv_, semd.at[b])

    @plsc.parallel_loop(0, NNP // 16, unroll=4)
    def _ep(g):
        off = g * 16
        dv = dinv_v[pl.ds(off, 16)]
        d2 = dv * dv
        for j in range(FPW):
            o = j * NNP + off
            acc[pl.ds(o, 16)] = acc[pl.ds(o, 16)] + d2 * xs[pl.ds(o, 16)]

    pltpu.sync_copy(acc, out.at[pl.ds(w * FPW * NNP, FPW * NNP)])


_spmm_full = functools.partial(
    pl.kernel,
    out_type=jax.ShapeDtypeStruct((D * NNP,), jnp.float32),
    mesh=_mesh,
    scratch_types=[
        pltpu.VMEM((FPW * NNP,), jnp.float32),
        pltpu.VMEM((FPW * NNP,), jnp.float32),
        pltpu.VMEM((NNP,), jnp.float32),
        pltpu.VMEM((ECH,), jnp.int32),
        pltpu.VMEM((ECH,), jnp.int32),
        pltpu.VMEM((ECH,), jnp.int32),
        pltpu.VMEM((ECH,), jnp.int32),
        pltpu.SemaphoreType.DMA((2,)),
        pltpu.SemaphoreType.DMA((2,)),
    ],
    compiler_params=_sc_params,
)(_spmm_body)


def _spmm_sel_body(hw_t, head, l2s, l2d, cnts, dinv, zeros, n_idx, out,
                   xs, acc, dinv_v, head_v, cv, ovs, ovd, n_v, sel_v):
    w = _wid()
    pltpu.sync_copy(hw_t.at[pl.ds(w * FPW * NNP, FPW * NNP)], xs)
    pltpu.sync_copy(dinv, dinv_v)
    pltpu.sync_copy(zeros, acc)
    pltpu.sync_copy(head, head_v)
    pltpu.sync_copy(cnts, cv)

    def _edge_group(sref, sbase, dref, dbase, g):
        s16 = sref[pl.ds(sbase + g * 16, 16)]
        d16 = dref[pl.ds(dbase + g * 16, 16)]
        nv = (plsc.load_gather(dinv_v, [s16])
              * plsc.load_gather(dinv_v, [d16]))
        for j in range(FPW):
            xv = plsc.load_gather(xs, [s16 + (j * NNP)])
            plsc.addupdate_scatter(acc, [d16 + (j * NNP)], xv * nv)

    for v in range(NW):
        m = jnp.sum(cv[pl.ds(v * 16, 16)])
        mm = jnp.minimum(m, HCAP)
        hb = v * 2 * HCAP

        def grp(g, c, hb=hb):
            _edge_group(head_v, hb, head_v, hb + HCAP, g)
            return c

        lax.fori_loop(0, (mm + 15) // 16, grp, 0)

    def ov(v, c):
        m = jnp.sum(cv[pl.ds(v * 16, 16)])

        @pl.when(m > HCAP)
        def _():
            def part(p, c2):
                off0 = v * EPW + p * PCH
                pltpu.sync_copy(l2s.at[pl.ds(off0, PCH)], ovs)
                pltpu.sync_copy(l2d.at[pl.ds(off0, PCH)], ovd)
                rem = jnp.minimum(m - p * PCH, PCH)

                def grp2(g, c3):
                    _edge_group(ovs, 0, ovd, 0, g)
                    return c3

                lax.fori_loop(0, (rem + 15) // 16, grp2, 0)
                return c2

            lax.fori_loop(1, (m + PCH - 1) // PCH, part, 0)

        return c

    lax.fori_loop(0, NW, ov, 0)

    pltpu.sync_copy(n_idx, n_v)
    for g in range(T // 16):
        t16 = n_v[pl.ds(g * 16, 16)]
        dv = plsc.load_gather(dinv_v, [t16])
        d2 = dv * dv
        for j in range(FPW):
            av = plsc.load_gather(acc, [t16 + (j * NNP)])
            xv = plsc.load_gather(xs, [t16 + (j * NNP)])
            sel_v[pl.ds(j * T + g * 16, 16)] = av + d2 * xv
    pltpu.sync_copy(sel_v, out.at[pl.ds(w * FPW * T, FPW * T)])


_spmm_sel = functools.partial(
    pl.kernel,
    out_type=jax.ShapeDtypeStruct((NW * FPW * T,), jnp.float32),
    mesh=_mesh,
    scratch_types=[
        pltpu.VMEM((FPW * NNP,), jnp.float32),
        pltpu.VMEM((FPW * NNP,), jnp.float32),
        pltpu.VMEM((NNP,), jnp.float32),
        pltpu.VMEM((NW * 2 * HCAP,), jnp.int32),
        pltpu.VMEM((NW * 16,), jnp.int32),
        pltpu.VMEM((PCH,), jnp.int32),
        pltpu.VMEM((PCH,), jnp.int32),
        pltpu.VMEM((T,), jnp.int32),
        pltpu.VMEM((FPW * T,), jnp.float32),
    ],
    compiler_params=_sc_params,
)(_spmm_sel_body)


def _ln_w1_body(rows_ref, w1_ref, out_ref):
    r = rows_ref[:]
    mu = jnp.mean(r, axis=-1, keepdims=True)
    var = jnp.mean((r - mu) ** 2, axis=-1, keepdims=True)
    x = (r - mu) * lax.rsqrt(var + 1e-5)
    out_ref[:] = lax.dot_general(
        w1_ref[:], x, (((1,), (1,)), ((), ())),
        preferred_element_type=jnp.float32)


def _dinv_body(degp_ref, out_ref):
    deg = jnp.sum(degp_ref[:], axis=0) + 1.0
    out_ref[:] = lax.rsqrt(deg)


def _act_w2_body(a_ref, b1_ref, w2_ref, out_ref):
    hb = a_ref[:] + b1_ref[:]
    hb = jnp.where(hb > 0, hb, 0.15 * hb)
    out_ref[:] = lax.dot_general(
        w2_ref[:], hb, (((1,), (0,)), ((), ())),
        preferred_element_type=jnp.float32)


def _head_body(sel_ref, b2_ref, wout_ref, bout_ref, out_ref):
    a = sel_ref[:] + b2_ref[:]
    out_ref[:] = lax.dot_general(
        a, wout_ref[:], (((0,), (1,)), ((), ())),
        preferred_element_type=jnp.float32) + bout_ref[:]


_NB = 1024
_VB = 2048


def kernel(edge_index, N, y, emb, W1, b1, W2, b2, Wout, bout):
    src = edge_index[0].astype(jnp.int32)
    dst = edge_index[1].astype(jnp.int32)
    y_pad = jnp.concatenate(
        [y.astype(jnp.int32), jnp.zeros((NNP - NN,), jnp.int32)]
    ).reshape(NW, NGC, GCH)
    n_idx = N.astype(jnp.int32)
    zeros = jnp.zeros((FPW * NNP,), jnp.float32)

    rows, deg_p, l2s, l2d, head, cnts = _prep(emb, y_pad, src, dst,
                                              n_idx, zeros)

    xw1_t = pl.pallas_call(
        _ln_w1_body,
        grid=(NNP // _NB,),
        in_specs=[
            pl.BlockSpec((_NB, D), lambda i: (i, 0)),
            pl.BlockSpec((D, D), lambda i: (0, 0)),
        ],
        out_specs=pl.BlockSpec((D, _NB), lambda i: (0, i)),
        out_shape=jax.ShapeDtypeStruct((D, NNP), jnp.float32),
    )(rows, W1)

    dinv = pl.pallas_call(
        _dinv_body,
        in_specs=[pl.BlockSpec((NW, NNP // D, D), lambda: (0, 0, 0))],
        out_specs=pl.BlockSpec((NNP // D, D), lambda: (0, 0)),
        out_shape=jax.ShapeDtypeStruct((NNP // D, D), jnp.float32),
    )(deg_p.reshape(NW, NNP // D, D)).reshape(NNP)

    a1_t = _spmm_full(xw1_t.reshape(D * NNP), src, dst, dinv, zeros)

    hw2_t = pl.pallas_call(
        _act_w2_body,
        grid=(NNP // _NB,),
        in_specs=[
            pl.BlockSpec((D, _NB), lambda i: (0, i)),
            pl.BlockSpec((D, 1), lambda i: (0, 0)),
            pl.BlockSpec((D, D), lambda i: (0, 0)),
        ],
        out_specs=pl.BlockSpec((D, _NB), lambda i: (0, i)),
        out_shape=jax.ShapeDtypeStruct((D, NNP), jnp.float32),
    )(a1_t.reshape(D, NNP), b1.reshape(D, 1), W2)

    sel = _spmm_sel(hw2_t.reshape(D * NNP), head, l2s, l2d, cnts,
                    dinv, zeros, n_idx)

    out = pl.pallas_call(
        _head_body,
        grid=(pl.cdiv(V, _VB),),
        in_specs=[
            pl.BlockSpec((D, T), lambda i: (0, 0)),
            pl.BlockSpec((D, 1), lambda i: (0, 0)),
            pl.BlockSpec((_VB, D), lambda i: (i, 0)),
            pl.BlockSpec((1, _VB), lambda i: (0, i)),
        ],
        out_specs=pl.BlockSpec((T, _VB), lambda i: (0, i)),
        out_shape=jax.ShapeDtypeStruct((T, V), jnp.float32),
    )(sel.reshape(NW * FPW, T).reshape(D, T), b2.reshape(D, 1),
      Wout, bout.reshape(1, V))

    return out
